# baseline (speedup 1.0000x reference)
import functools

import jax
import jax.numpy as jnp
from jax import lax
from jax.experimental import pallas as pl
from jax.experimental.pallas import tpu as pltpu
from jax.experimental.pallas import tpu_sc as plsc

_R = 16
_NB = 4
_PF = 3
_NW = 32
_NC = 2
_SC_ROWS = 3072
_TC_BLK = 512


@functools.lru_cache(maxsize=None)
def _make_sc_kernel(B, S, D, row_base, n_rows):
    rows_pw = n_rows // _NW
    n_chunks = rows_pw // _R
    T = n_chunks * B
    vpr = D // 16
    mesh = plsc.VectorSubcoreMesh(core_axis_name="c", subcore_axis_name="s")

    @functools.partial(
        pl.kernel,
        out_type=jax.ShapeDtypeStruct((B, n_rows, D), jnp.float32),
        mesh=mesh,
        scratch_types=[
            pltpu.VMEM((_NB * _R, D), jnp.float32),
            pltpu.VMEM((2 * _R, D), jnp.float32),
            [pltpu.SemaphoreType.DMA] * _NB,
            [pltpu.SemaphoreType.DMA] * _NB,
            [pltpu.SemaphoreType.DMA] * 2,
        ],
    )
    def sc_add(x_hbm, pos_hbm, out_hbm, xbuf, pbuf, in_sems, out_sems, p_sems):
        wid = lax.axis_index("s") * _NC + lax.axis_index("c")
        row0 = wid * rows_pw

        def load_x(t, i):
            c, b = divmod(t, B)
            return pltpu.async_copy(
                x_hbm.at[b, pl.ds(row_base + row0 + c * _R, _R)],
                xbuf.at[pl.ds(i * _R, _R)],
                in_sems[i],
            )

        def load_p(c):
            return pltpu.async_copy(
                pos_hbm.at[pl.ds(row_base + row0 + c * _R, _R)],
                pbuf.at[pl.ds((c % 2) * _R, _R)],
                p_sems[c % 2],
            )

        def store_x(t, i):
            c, b = divmod(t, B)
            return pltpu.async_copy(
                xbuf.at[pl.ds(i * _R, _R)],
                out_hbm.at[b, pl.ds(row0 + c * _R, _R)],
                out_sems[i],
            )

        shift = vpr.bit_length() - 1
        pdesc = [None, None]
        pdesc[0] = load_p(0)
        in_desc = [None] * _NB
        for t in range(min(_PF, T)):
            in_desc[t % _NB] = load_x(t, t % _NB)
        out_desc = [None] * _NB

        for t in range(T):
            i = t % _NB
            c, b = divmod(t, B)
            in_desc[i].wait()
            if b == 0:
                pdesc[c % 2].wait()
                if c + 1 < n_chunks:
                    pdesc[(c + 1) % 2] = load_p(c + 1)
            prow = (c % 2) * _R

            @plsc.parallel_loop(0, _R * vpr, step=1, unroll=8)
            def _(j):
                r = lax.shift_right_logical(j, shift)
                col = pl.multiple_of(
                    lax.shift_left(lax.bitwise_and(j, vpr - 1), 4), 16
                )
                plsc.addupdate(
                    xbuf.at[i * _R + r, pl.ds(col, 16)],
                    pbuf[prow + r, pl.ds(col, 16)],
                )

            out_desc[i] = store_x(t, i)
            nt = t + _PF
            if nt < T:
                nb = nt % _NB
                if out_desc[nb] is not None:
                    out_desc[nb].wait()
                    out_desc[nb] = None
                in_desc[nb] = load_x(nt, nb)

        for i in range(_NB):
            if out_desc[i] is not None:
                out_desc[i].wait()

    return sc_add


def _tc_body(x_ref, p_ref, o_ref):
    o_ref[...] = x_ref[...] + p_ref[...]


def _stitch_body(tc_ref, sc_ref, o_ref):
    o_ref[...] = sc_ref[...]


@functools.lru_cache(maxsize=None)
def _make_tc_kernel(B, S, D, n_tc_rows):
    grid = (n_tc_rows // _TC_BLK, B)
    return pl.pallas_call(
        _tc_body,
        grid=grid,
        in_specs=[
            pl.BlockSpec((1, _TC_BLK, D), lambda s, b: (b, s, 0)),
            pl.BlockSpec((_TC_BLK, D), lambda s, b: (s, 0)),
        ],
        out_specs=pl.BlockSpec((1, _TC_BLK, D), lambda s, b: (b, s, 0)),
        out_shape=jax.ShapeDtypeStruct((B, S, D), jnp.float32),
    )


@functools.lru_cache(maxsize=None)
def _make_stitch_kernel(B, S, D, row_base, n_rows):
    blk0 = row_base // _TC_BLK
    grid = (n_rows // _TC_BLK, B)
    return pl.pallas_call(
        _stitch_body,
        grid=grid,
        in_specs=[
            pl.BlockSpec(memory_space=pl.ANY),
            pl.BlockSpec((1, _TC_BLK, D), lambda s, b: (b, s, 0)),
        ],
        out_specs=pl.BlockSpec((1, _TC_BLK, D), lambda s, b: (b, s + blk0, 0)),
        out_shape=jax.ShapeDtypeStruct((B, S, D), jnp.float32),
        input_output_aliases={0: 0},
    )


def kernel(x, pos_emb):
    B, S, D = x.shape
    sc_rows = _SC_ROWS
    tc_rows = S - sc_rows
    sc_out = _make_sc_kernel(B, S, D, tc_rows, sc_rows)(x, pos_emb)
    tc_out = _make_tc_kernel(B, S, D, tc_rows)(x, pos_emb)
    return _make_stitch_kernel(B, S, D, tc_rows, sc_rows)(tc_out, sc_out)


# speedup vs baseline: 1.1757x; 1.1757x over previous
import functools

import jax
import jax.numpy as jnp
from jax import lax
from jax.experimental import pallas as pl
from jax.experimental.pallas import tpu as pltpu
from jax.experimental.pallas import tpu_sc as plsc

_R = 16
_NB = 5
_PF = 4
_NW = 32
_NC = 2


@functools.lru_cache(maxsize=None)
def _make_sc_kernel(B, S, D):
    rows_pw = S // _NW
    n_chunks = rows_pw // _R
    T = n_chunks * B
    vpr = D // 16
    mesh = plsc.VectorSubcoreMesh(core_axis_name="c", subcore_axis_name="s")

    @functools.partial(
        pl.kernel,
        out_type=jax.ShapeDtypeStruct((B, S, D), jnp.float32),
        mesh=mesh,
        scratch_types=[
            pltpu.VMEM((_NB * _R, D), jnp.float32),
            pltpu.VMEM((2 * _R, D), jnp.float32),
            [pltpu.SemaphoreType.DMA] * _NB,
            [pltpu.SemaphoreType.DMA] * _NB,
            [pltpu.SemaphoreType.DMA] * 2,
        ],
    )
    def sc_add(x_hbm, pos_hbm, out_hbm, xbuf, pbuf, in_sems, out_sems, p_sems):
        wid = lax.axis_index("s") * _NC + lax.axis_index("c")
        row0 = wid * rows_pw

        def load_x(t, i):
            c, b = divmod(t, B)
            return pltpu.async_copy(
                x_hbm.at[b, pl.ds(row0 + c * _R, _R)],
                xbuf.at[pl.ds(i * _R, _R)],
                in_sems[i],
            )

        def load_p(c):
            return pltpu.async_copy(
                pos_hbm.at[pl.ds(row0 + c * _R, _R)],
                pbuf.at[pl.ds((c % 2) * _R, _R)],
                p_sems[c % 2],
            )

        def store_x(t, i):
            c, b = divmod(t, B)
            return pltpu.async_copy(
                xbuf.at[pl.ds(i * _R, _R)],
                out_hbm.at[b, pl.ds(row0 + c * _R, _R)],
                out_sems[i],
            )

        shift = vpr.bit_length() - 1
        pdesc = [None, None]
        pdesc[0] = load_p(0)
        in_desc = [None] * _NB
        for t in range(min(_PF, T)):
            in_desc[t % _NB] = load_x(t, t % _NB)
        out_desc = [None] * _NB

        for t in range(T):
            i = t % _NB
            c, b = divmod(t, B)
            in_desc[i].wait()
            if b == 0:
                pdesc[c % 2].wait()
                if c + 1 < n_chunks:
                    pdesc[(c + 1) % 2] = load_p(c + 1)
            prow = (c % 2) * _R

            @plsc.parallel_loop(0, _R * vpr, step=1, unroll=8)
            def _(j):
                r = lax.shift_right_logical(j, shift)
                col = pl.multiple_of(
                    lax.shift_left(lax.bitwise_and(j, vpr - 1), 4), 16
                )
                plsc.addupdate(
                    xbuf.at[i * _R + r, pl.ds(col, 16)],
                    pbuf[prow + r, pl.ds(col, 16)],
                )

            out_desc[i] = store_x(t, i)
            nt = t + _PF
            if nt < T:
                nb = nt % _NB
                if out_desc[nb] is not None:
                    out_desc[nb].wait()
                    out_desc[nb] = None
                in_desc[nb] = load_x(nt, nb)

        for i in range(_NB):
            if out_desc[i] is not None:
                out_desc[i].wait()

    return sc_add


def kernel(x, pos_emb):
    B, S, D = x.shape
    return _make_sc_kernel(B, S, D)(x, pos_emb)
